# baseline (speedup 1.0000x reference)
import functools

import jax
import jax.numpy as jnp
from jax import lax
from jax.experimental import pallas as pl
from jax.experimental.pallas import tpu as pltpu
from jax.experimental.pallas import tpu_sc as plsc

NBUF = 4
CHUNK = 8
NBSC = 2
B, A, D = 16, 50000, 8
STRIP = 6272
CH = 1536


def _tc_body(lp_hbm, lt_hbm, x_ref, y_ref, out_ref, pt_ref,
             lpb, ltb, pos_ref, lpw, ltw, sems):
    a = y_ref.shape[1]
    nchunks = (B - NBSC) * D // CHUNK

    def _copies(c, slot):
        return [
            pltpu.make_async_copy(
                lp_hbm.at[pl.ds(c * CHUNK, CHUNK), :], lpb.at[slot],
                sems.at[slot, 0]),
            pltpu.make_async_copy(
                lt_hbm.at[pl.ds(c * CHUNK, CHUNK), :], ltb.at[slot],
                sems.at[slot, 1]),
        ]

    def start(c, slot):
        for cp in _copies(c, slot):
            cp.start()

    def wait(c, slot):
        for cp in _copies(c, slot):
            cp.wait()

    for c in range(min(NBUF, nchunks)):
        start(c, c)

    y = y_ref[...]
    lane = lax.broadcasted_iota(jnp.int32, y.shape, 1)
    valid = lane < a
    t = (valid & (y == 1)).astype(jnp.float32)
    pos = (valid & (y > 0)).astype(jnp.float32)
    x = x_ref[...].reshape(y.shape)
    z = 2.0 * x * (2.0 * t - 1.0) + 1.0
    neg_logpt = jnp.log(1.0 + jnp.exp(-jnp.abs(z))) - jnp.minimum(z, 0.0)
    w = 0.75 - 0.5 * t
    cls_elem = jnp.where(valid & (y > -1), w * neg_logpt, 0.0)
    cls_sum = 0.5 * jnp.sum(cls_elem)
    np_sum = jnp.sum(pos)
    pos3 = pos.reshape(pos_ref.shape)
    pos_ref[...] = pos3
    pt_ref[...] = pos3[B - NBSC:]

    def chunk_body(c, acc):
        slot = lax.rem(c, NBUF)
        wait(c, slot)
        d = lpb[slot] - ltb[slot]
        nxt = c + NBUF

        @pl.when(nxt < nchunks)
        def _():
            start(nxt, lax.rem(nxt, NBUF))

        ad = jnp.abs(d)
        q = jnp.minimum(ad, 1.0)
        sl1 = q * (ad - 0.5 * q)
        rs = jnp.sum(sl1.reshape(-1, D, a), axis=1)
        lane1 = lax.broadcasted_iota(jnp.int32, rs.shape, 1)
        rs = jnp.where(lane1 < a, rs, 0.0)
        bb = rs.shape[0]
        pr = pos_ref[pl.ds(c * bb, bb), 0, :]
        return acc + jnp.sum(rs * pr)

    loc_sum = lax.fori_loop(0, nchunks, chunk_body, 0.0, unroll=False)

    w0 = (a // 128) * 128
    cw1 = pltpu.make_async_copy(
        lp_hbm.at[pl.ds((B - NBSC) * D, NBSC * D), pl.ds(w0, a - w0)],
        lpw, sems.at[0, 2])
    cw2 = pltpu.make_async_copy(
        lt_hbm.at[pl.ds((B - NBSC) * D, NBSC * D), pl.ds(w0, a - w0)],
        ltw, sems.at[1, 2])
    cw1.start()
    cw2.start()
    cw1.wait()
    cw2.wait()
    dw = lpw[...] - ltw[...]
    adw = jnp.abs(dw)
    qw = jnp.minimum(adw, 1.0)
    sw = qw * (adw - 0.5 * qw)
    rsw = jnp.sum(sw.reshape(NBSC, D, a - w0), axis=1)
    posw = pos_ref[pl.ds(B - NBSC, NBSC), 0, pl.ds(w0, a - w0)]
    loc_sum = loc_sum + jnp.sum(rsw * posw)

    out_ref[0] = loc_sum
    out_ref[1] = cls_sum
    out_ref[2] = np_sum


def _tail_body(rs_ref, pt_ref, out_ref):
    rs = rs_ref[...]
    pt = pt_ref[...]
    lane = lax.broadcasted_iota(jnp.int32, rs.shape, 2)
    out_ref[0] = jnp.sum(jnp.where(lane < (A // 128) * 128, rs * pt, 0.0))


def _sc_worker_math(bufp, buft, accv, ch, nvec):
    def vbody(vi, _):
        acc = jnp.zeros((16,), jnp.float32)
        for dd in range(D):
            dv = bufp[dd, pl.ds(vi * 16, 16)] - buft[dd, pl.ds(vi * 16, 16)]
            ad = jnp.abs(dv)
            q = jnp.minimum(ad, 1.0)
            acc = acc + q * (ad - 0.5 * q)
        accv[0, pl.ds(vi * 16, 16)] = acc
        return 0

    lax.fori_loop(0, nvec, vbody, 0)


def _sc_chunk(lp_hbm, lt_hbm, out_hbm, bufp, buft, accv, sem1, sem2,
              row0, bi, off, ch):
    row0 = pl.multiple_of(row0, 8)
    off = pl.multiple_of(off, 128)
    c1 = pltpu.async_copy(
        lp_hbm.at[pl.ds(row0, D), pl.ds(off, ch)],
        bufp.at[:, pl.ds(0, ch)], sem1)
    c2 = pltpu.async_copy(
        lt_hbm.at[pl.ds(row0, D), pl.ds(off, ch)],
        buft.at[:, pl.ds(0, ch)], sem2)
    c1.wait()
    c2.wait()
    _sc_worker_math(bufp, buft, accv, ch, ch // 16)
    pltpu.sync_copy(accv.at[:, pl.ds(0, ch)], out_hbm.at[bi, :, pl.ds(off, ch)])


def _make_sc(info):
    scmesh = plsc.VectorSubcoreMesh(core_axis_name="c", subcore_axis_name="s")

    @functools.partial(
        pl.kernel, mesh=scmesh,
        out_type=jax.ShapeDtypeStruct((NBSC, 1, A), jnp.float32),
        scratch_types=[
            pltpu.VMEM((D, CH), jnp.float32),
            pltpu.VMEM((D, CH), jnp.float32),
            pltpu.VMEM((1, CH), jnp.float32),
            pltpu.SemaphoreType.DMA,
            pltpu.SemaphoreType.DMA,
        ],
    )
    def sc_rs(lp_hbm, lt_hbm, out_hbm, bufp, buft, accv, sem1, sem2):
        wid = lax.axis_index("s") * info.num_cores + lax.axis_index("c")
        bi = wid // 8
        s = lax.rem(wid, 8)
        b = (B - NBSC) + bi
        row0 = b * D
        off0 = jnp.where(s < 6, s * STRIP, 6 * STRIP + (s - 6) * (4 * CH))

        def body(ci, _):
            _sc_chunk(lp_hbm, lt_hbm, out_hbm, bufp, buft, accv, sem1, sem2,
                      row0, bi, off0 + ci * CH, CH)
            return 0

        lax.fori_loop(0, 4, body, 0)

        @pl.when(s < 6)
        def _tail_small():
            _sc_chunk(lp_hbm, lt_hbm, out_hbm, bufp, buft, accv, sem1, sem2,
                      row0, bi, off0 + 4 * CH, 128)

    return sc_rs


def kernel(loc_preds, loc_targets, cls_preds, cls_targets):
    b, a, dd = loc_preds.shape
    lp = jnp.transpose(loc_preds, (0, 2, 1)).reshape(b * dd, a)
    lt = jnp.transpose(loc_targets, (0, 2, 1)).reshape(b * dd, a)
    x = jnp.transpose(cls_preds, (0, 2, 1))
    y = cls_targets

    info = plsc.get_sparse_core_info()
    rs = _make_sc(info)(lp, lt)

    scalars, pos_tail = pl.pallas_call(
        _tc_body,
        in_specs=[
            pl.BlockSpec(memory_space=pl.ANY),
            pl.BlockSpec(memory_space=pl.ANY),
            pl.BlockSpec((b, 1, a), lambda: (0, 0, 0)),
            pl.BlockSpec((b, a), lambda: (0, 0)),
        ],
        out_specs=[
            pl.BlockSpec(memory_space=pltpu.SMEM),
            pl.BlockSpec((NBSC, 1, a), lambda: (0, 0, 0)),
        ],
        out_shape=[
            jax.ShapeDtypeStruct((3,), jnp.float32),
            jax.ShapeDtypeStruct((NBSC, 1, a), jnp.float32),
        ],
        scratch_shapes=[
            pltpu.VMEM((NBUF, CHUNK, a), jnp.float32),
            pltpu.VMEM((NBUF, CHUNK, a), jnp.float32),
            pltpu.VMEM((b, 1, a), jnp.float32),
            pltpu.VMEM((NBSC * dd, a - (a // 128) * 128), jnp.float32),
            pltpu.VMEM((NBSC * dd, a - (a // 128) * 128), jnp.float32),
            pltpu.SemaphoreType.DMA((NBUF, 3)),
        ],
    )(lp, lt, x, y)

    loc_sc = pl.pallas_call(
        _tail_body,
        in_specs=[
            pl.BlockSpec((NBSC, 1, a), lambda: (0, 0, 0)),
            pl.BlockSpec((NBSC, 1, a), lambda: (0, 0, 0)),
        ],
        out_specs=pl.BlockSpec(memory_space=pltpu.SMEM),
        out_shape=jax.ShapeDtypeStruct((1,), jnp.float32),
    )(rs, pos_tail)

    loc_sum = scalars[0] + loc_sc[0]
    return (0.2 * loc_sum + scalars[1]) / scalars[2]


# speedup vs baseline: 1.6521x; 1.6521x over previous
import jax
import jax.numpy as jnp
from jax import lax
from jax.experimental import pallas as pl
from jax.experimental.pallas import tpu as pltpu

NBUF = 4
CHUNK = 8


def _body(lp_hbm, lt_hbm, x_ref, y_ref, out_ref,
          lpb, ltb, pos_ref, acc_ref, sems):
    a = y_ref.shape[1]
    nchunks = lp_hbm.shape[0] // CHUNK

    y = y_ref[...]
    lane = lax.broadcasted_iota(jnp.int32, y.shape, 1)
    valid = lane < a
    t = (valid & (y == 1)).astype(jnp.float32)
    pos = (valid & (y > 0)).astype(jnp.float32)
    x = x_ref[...].reshape(y.shape)
    z = 2.0 * x * (2.0 * t - 1.0) + 1.0
    neg_logpt = jnp.log(1.0 + jnp.exp(-jnp.abs(z))) - jnp.minimum(z, 0.0)
    w = 0.75 - 0.5 * t
    cls_elem = jnp.where(valid & (y > -1), w * neg_logpt, 0.0)
    cls_sum = 0.5 * jnp.sum(cls_elem)
    np_sum = jnp.sum(pos)
    pos_ref[...] = pos.reshape(pos_ref.shape)

    def start(c, slot):
        pltpu.make_async_copy(
            lp_hbm.at[pl.ds(c * CHUNK, CHUNK), :], lpb.at[slot], sems.at[slot, 0]
        ).start()
        pltpu.make_async_copy(
            lt_hbm.at[pl.ds(c * CHUNK, CHUNK), :], ltb.at[slot], sems.at[slot, 1]
        ).start()

    def wait(c, slot):
        pltpu.make_async_copy(
            lp_hbm.at[pl.ds(c * CHUNK, CHUNK), :], lpb.at[slot], sems.at[slot, 0]
        ).wait()
        pltpu.make_async_copy(
            lt_hbm.at[pl.ds(c * CHUNK, CHUNK), :], ltb.at[slot], sems.at[slot, 1]
        ).wait()

    for c in range(min(NBUF, nchunks)):
        start(c, c)

    def chunk_body(c, acc):
        slot = lax.rem(c, NBUF)
        wait(c, slot)
        d = lpb[slot] - ltb[slot]
        nxt = c + NBUF

        @pl.when(nxt < nchunks)
        def _():
            start(nxt, lax.rem(nxt, NBUF))

        ad = jnp.abs(d)
        q = jnp.minimum(ad, 1.0)
        sl1 = q * (ad - 0.5 * q)
        rs = jnp.sum(sl1.reshape(-1, 8, a), axis=1)
        lane1 = lax.broadcasted_iota(jnp.int32, rs.shape, 1)
        rs = jnp.where(lane1 < a, rs, 0.0)
        bb = rs.shape[0]
        pr = pos_ref[pl.ds(c * bb, bb), 0, :]
        return acc + jnp.sum(rs * pr)

    loc_sum = lax.fori_loop(0, nchunks, chunk_body, 0.0, unroll=False)
    acc_ref[0] = loc_sum
    inv = 1.0 / np_sum
    out_ref[0] = (0.2 * loc_sum + cls_sum) * inv


def kernel(loc_preds, loc_targets, cls_preds, cls_targets):
    b, a, dd = loc_preds.shape
    lp = jnp.transpose(loc_preds, (0, 2, 1)).reshape(b * dd, a)
    lt = jnp.transpose(loc_targets, (0, 2, 1)).reshape(b * dd, a)
    x = jnp.transpose(cls_preds, (0, 2, 1))
    y = cls_targets

    out = pl.pallas_call(
        _body,
        in_specs=[
            pl.BlockSpec(memory_space=pl.ANY),
            pl.BlockSpec(memory_space=pl.ANY),
            pl.BlockSpec((b, 1, a), lambda: (0, 0, 0)),
            pl.BlockSpec((b, a), lambda: (0, 0)),
        ],
        out_specs=pl.BlockSpec(memory_space=pltpu.SMEM),
        out_shape=jax.ShapeDtypeStruct((1,), jnp.float32),
        scratch_shapes=[
            pltpu.VMEM((NBUF, CHUNK, a), jnp.float32),
            pltpu.VMEM((NBUF, CHUNK, a), jnp.float32),
            pltpu.VMEM((b, 1, a), jnp.float32),
            pltpu.SMEM((1,), jnp.float32),
            pltpu.SemaphoreType.DMA((NBUF, 2)),
        ],
    )(lp, lt, x, y)
    return out[0]
